# baseline (device time: 33222 ns/iter reference)
import jax
import jax.numpy as jnp
from jax import lax
from jax.experimental import pallas as pl
from jax.experimental.pallas import tpu as pltpu

N_DEV = 4
B, Sq, Hq, Dh = 2, 128, 4, 64
BLK = 64


def kernel(x, Wq, K_ext, V_ext, Wo):
    skv_loc = K_ext.shape[1]
    d_model = x.shape[-1]

    def body(x_ref, wq_ref, k_ref, v_ref, wo_ref, out_ref,
             kv_buf, send_sems, recv_sem):
        my = lax.axis_index("i")

        barrier = pltpu.get_barrier_semaphore()

        @pl.when(my == 0)
        def _():
            for d in range(1, N_DEV):
                pl.semaphore_signal(
                    barrier, inc=1, device_id=(d,),
                    device_id_type=pl.DeviceIdType.MESH)
            pl.semaphore_wait(barrier, N_DEV - 1)

        @pl.when(my != 0)
        def _():
            pl.semaphore_signal(
                barrier, inc=1, device_id=(0,),
                device_id_type=pl.DeviceIdType.MESH)
            pl.semaphore_wait(barrier, 1)

        @pl.when(my == 0)
        def _():
            kv_buf[0, ...] = k_ref[...]
            kv_buf[1, ...] = v_ref[...]
            rdmas = []
            for idx, d in enumerate(range(1, N_DEV)):
                rdma = pltpu.make_async_remote_copy(
                    src_ref=kv_buf,
                    dst_ref=kv_buf,
                    send_sem=send_sems.at[idx],
                    recv_sem=recv_sem,
                    device_id=(d,),
                    device_id_type=pl.DeviceIdType.MESH,
                )
                rdma.start()
                rdmas.append(rdma)
            for rdma in rdmas:
                rdma.wait_send()

        @pl.when(my != 0)
        def _():
            recv = pltpu.make_async_remote_copy(
                src_ref=kv_buf,
                dst_ref=kv_buf,
                send_sem=send_sems.at[0],
                recv_sem=recv_sem,
                device_id=(0,),
                device_id_type=pl.DeviceIdType.MESH,
            )
            recv.wait_recv()

        wq = wq_ref[...]
        wo = wo_ref[...]
        qb = lax.broadcasted_iota(jnp.int32, (Sq, skv_loc), 0) // BLK
        kb = lax.broadcasted_iota(jnp.int32, (Sq, skv_loc), 1) // BLK
        mask = kb <= qb
        for b in range(B):
            xb = x_ref[b]
            q_all = jnp.dot(xb, wq, preferred_element_type=jnp.float32)
            ctx_heads = []
            for h in range(Hq):
                q_bh = q_all[:, h * Dh:(h + 1) * Dh]
                k_bh = kv_buf[0, b, :, h, :]
                v_bh = kv_buf[1, b, :, h, :]
                s = lax.dot_general(
                    q_bh, k_bh, (((1,), (1,)), ((), ())),
                    preferred_element_type=jnp.float32) * 0.125
                s = jnp.where(mask, s, -1e9)
                m = jnp.max(s, axis=1, keepdims=True)
                w = jnp.exp(s - m)
                w = w / jnp.sum(w, axis=1, keepdims=True)
                ctx_heads.append(jnp.dot(
                    w, v_bh, preferred_element_type=jnp.float32))
            ctx = jnp.concatenate(ctx_heads, axis=1)
            out_ref[b] = jnp.dot(
                ctx, wo, preferred_element_type=jnp.float32)

    return pl.pallas_call(
        body,
        out_shape=jax.ShapeDtypeStruct((B, Sq, d_model), jnp.float32),
        in_specs=[pl.BlockSpec(memory_space=pltpu.VMEM)] * 5,
        out_specs=pl.BlockSpec(memory_space=pltpu.VMEM),
        scratch_shapes=[
            pltpu.VMEM((2, B, skv_loc, Hq, Dh), jnp.float32),
            pltpu.SemaphoreType.DMA((N_DEV - 1,)),
            pltpu.SemaphoreType.DMA,
        ],
        compiler_params=pltpu.CompilerParams(collective_id=0),
    )(x, Wq, K_ext, V_ext, Wo)
